# baseline (speedup 1.0000x reference)
import functools

import jax
import jax.numpy as jnp
from jax import lax
from jax.experimental import pallas as pl
from jax.experimental.pallas import tpu as pltpu
from jax.experimental.pallas import tpu_sc as plsc

N = 10000
D_IN = 128
H = 64
Z = 32
E = 320000

NC = 2
NS = 16
NW = NC * NS

K = 128
CH = 80
CH0 = 40
CH1 = 2 * CH - CH0
CHMX = max(CH0, CH1)
E_PAD = NW * CH * K
N_PAD = 10112
RPT = N_PAD // NS
DUMMY = N

def _mesh():
    return plsc.VectorSubcoreMesh(
        core_axis_name="c", subcore_axis_name="s", num_cores=NC, num_subcores=NS
    )


def _sc_deg(dst2d, ones_h, zeros_h):
    @functools.partial(
        pl.kernel,
        out_type=jax.ShapeDtypeStruct((NC, NS, RPT), jnp.float32),
        mesh=_mesh(),
        scratch_types=[
            pltpu.VMEM((CH, K), jnp.int32),
            pltpu.VMEM((K,), jnp.float32),
            pltpu.VMEM((RPT,), jnp.float32),
            pltpu.VMEM_SHARED((N_PAD,), jnp.float32),
        ],
        name="sc_deg",
    )
    def body(dst_h, ones_hbm, zeros_hbm, out_h, didx, ones_v, stage, acc):
        c = lax.axis_index("c")
        s = lax.axis_index("s")
        wid = s * NC + c
        pltpu.sync_copy(zeros_hbm.at[pl.ds(s * RPT, RPT)], stage)
        pltpu.sync_copy(stage, acc.at[pl.ds(s * RPT, RPT)])
        pltpu.sync_copy(ones_hbm, ones_v)
        pltpu.sync_copy(dst_h.at[pl.ds(wid * CH, CH)], didx)
        plsc.subcore_barrier()

        def step(j, carry):
            pltpu.sync_copy(ones_v, acc.at[didx.at[j]], add=True)
            return carry

        lax.fori_loop(0, CH, step, 0)
        plsc.subcore_barrier()
        pltpu.sync_copy(acc.at[pl.ds(s * RPT, RPT)], stage)
        pltpu.sync_copy(stage, out_h.at[c, s])

    return body(dst2d, ones_h, zeros_h)


def _sc_gather_scatter(hs, src2d, dst2d, zeros_h, d):
    @functools.partial(
        pl.kernel,
        out_type=jax.ShapeDtypeStruct((NC, NS, RPT, d), jnp.float32),
        mesh=_mesh(),
        scratch_types=[
            pltpu.VMEM((CHMX, K), jnp.int32),
            pltpu.VMEM((CHMX, K), jnp.int32),
            pltpu.VMEM((K, d), jnp.float32),
            pltpu.VMEM((K, d), jnp.float32),
            pltpu.VMEM((RPT, d), jnp.float32),
            pltpu.VMEM_SHARED((N_PAD, d), jnp.float32),
            pltpu.SemaphoreType.DMA,
            pltpu.SemaphoreType.DMA,
        ],
        compiler_params=pltpu.CompilerParams(use_tc_tiling_on_sc=False),
        name=f"sc_gs{d}",
    )
    def body(hs_h, src_h, dst_h, zeros_hbm, out_h,
             sidx, didx, rows0, rows1, stage, acc, sem0, sem1):
        c = lax.axis_index("c")
        s = lax.axis_index("s")
        pltpu.sync_copy(zeros_hbm.at[pl.ds(s * RPT, RPT)], stage)
        pltpu.sync_copy(stage, acc.at[pl.ds(s * RPT, RPT)])

        def run(ch, base):
            pltpu.sync_copy(src_h.at[pl.ds(base, ch)], sidx.at[pl.ds(0, ch)])
            pltpu.sync_copy(dst_h.at[pl.ds(base, ch)], didx.at[pl.ds(0, ch)])
            pltpu.async_copy(hs_h.at[sidx.at[0]], rows0, sem0)
            pltpu.async_copy(hs_h.at[sidx.at[1]], rows1, sem1)
            nh = ch // 2

            def step(jj, carry):
                j0 = 2 * jj

                def half(rows, sem, j):
                    pltpu.make_async_copy(hs_h.at[sidx.at[j]], rows, sem).wait()
                    pltpu.sync_copy(rows, acc.at[didx.at[j]], add=True)

                    @pl.when(jj < nh - 1)
                    def _():
                        pltpu.async_copy(hs_h.at[sidx.at[j + 2]], rows, sem)

                half(rows0, sem0, j0)
                half(rows1, sem1, j0 + 1)
                return carry

            lax.fori_loop(0, nh, step, 0)

        @pl.when(c == 0)
        def _():
            run(CH0, s * (CH0 + CH1))

        @pl.when(c == 1)
        def _():
            run(CH1, s * (CH0 + CH1) + CH0)

        plsc.subcore_barrier()
        pltpu.sync_copy(acc.at[pl.ds(s * RPT, RPT)], stage)
        pltpu.sync_copy(stage, out_h.at[c, s])

    return body(hs, src2d, dst2d, zeros_h)


_GB = 16
_BR = N_PAD // _GB


def _tc_scale_mm1(x_pad, w1, degp):
    def body(deg_ref, x_ref, w_ref, hs_ref, dinv_ref):
        deg = deg_ref[:, 0] + deg_ref[:, 1] + 1.0
        dinv = lax.rsqrt(jnp.maximum(deg, 1e-12))
        h = jnp.dot(x_ref[...], w_ref[...], preferred_element_type=jnp.float32)
        hs_ref[...] = h * dinv[:, None]
        dinv_ref[...] = dinv[:, None]

    return pl.pallas_call(
        body,
        grid=(_GB,),
        in_specs=[
            pl.BlockSpec((_BR, NC), lambda i: (i, 0)),
            pl.BlockSpec((_BR, D_IN), lambda i: (i, 0)),
            pl.BlockSpec((D_IN, H), lambda i: (0, 0)),
        ],
        out_specs=[
            pl.BlockSpec((_BR, H), lambda i: (i, 0)),
            pl.BlockSpec((_BR, 1), lambda i: (i, 0)),
        ],
        out_shape=[
            jax.ShapeDtypeStruct((N_PAD, H), jnp.float32),
            jax.ShapeDtypeStruct((N_PAD, 1), jnp.float32),
        ],
    )(degp.T, x_pad, w1)


def _tc_relu_mm2(acc1, hs1, dinv, w_mu):
    def body(a_ref, hs_ref, dinv_ref, w_ref, out_ref):
        dv = dinv_ref[...]
        h2 = jnp.maximum((a_ref[0] + a_ref[1] + hs_ref[...]) * dv, 0.0)
        out_ref[...] = (
            jnp.dot(h2, w_ref[...], preferred_element_type=jnp.float32) * dv
        )

    return pl.pallas_call(
        body,
        grid=(_GB,),
        in_specs=[
            pl.BlockSpec((NC, _BR, H), lambda i: (0, i, 0)),
            pl.BlockSpec((_BR, H), lambda i: (i, 0)),
            pl.BlockSpec((_BR, 1), lambda i: (i, 0)),
            pl.BlockSpec((H, Z), lambda i: (0, 0)),
        ],
        out_specs=pl.BlockSpec((_BR, Z), lambda i: (i, 0)),
        out_shape=jax.ShapeDtypeStruct((N_PAD, Z), jnp.float32),
    )(acc1, hs1, dinv, w_mu)


def _tc_mu(acc2, hs2, dinv):
    def body(a_ref, hs_ref, dinv_ref, out_ref):
        i = pl.program_id(0)
        mu = (a_ref[0] + a_ref[1] + hs_ref[...]) * dinv_ref[...]
        rows = lax.broadcasted_iota(jnp.int32, (_BR, Z), 0) + i * _BR
        out_ref[...] = jnp.where(rows < N, mu, 0.0)

    return pl.pallas_call(
        body,
        grid=(_GB,),
        in_specs=[
            pl.BlockSpec((NC, _BR, Z), lambda i: (0, i, 0)),
            pl.BlockSpec((_BR, Z), lambda i: (i, 0)),
            pl.BlockSpec((_BR, 1), lambda i: (i, 0)),
        ],
        out_specs=pl.BlockSpec((_BR, Z), lambda i: (i, 0)),
        out_shape=jax.ShapeDtypeStruct((N_PAD, Z), jnp.float32),
    )(acc2, hs2, dinv)


_PAD_CORR = 0.5 * float(N_PAD * N_PAD - N * N)


def _tc_decode(mu):
    def body(zi_ref, zj_ref, out_ref, acc_ref):
        i = pl.program_id(0)
        j = pl.program_id(1)

        @pl.when(jnp.logical_and(i == 0, j == 0))
        def _init():
            acc_ref[0, 0] = 0.0

        logits = lax.dot_general(
            zi_ref[...], zj_ref[...], (((1,), (1,)), ((), ())),
            preferred_element_type=jnp.float32,
        )
        acc_ref[0, 0] += jnp.sum(jax.nn.sigmoid(logits))

        @pl.when(jnp.logical_and(i == _GB - 1, j == _GB - 1))
        def _fin():
            out_ref[0, 0] = -(acc_ref[0, 0] - _PAD_CORR) / float(N * N)

    return pl.pallas_call(
        body,
        grid=(_GB, _GB),
        in_specs=[
            pl.BlockSpec((_BR, Z), lambda i, j: (i, 0)),
            pl.BlockSpec((_BR, Z), lambda i, j: (j, 0)),
        ],
        out_specs=pl.BlockSpec(memory_space=pltpu.SMEM),
        out_shape=jax.ShapeDtypeStruct((1, 1), jnp.float32),
        scratch_shapes=[pltpu.SMEM((1, 1), jnp.float32)],
    )(mu, mu)


@jax.jit
def kernel(feature, edge_index, W1, W_mu):
    src = edge_index[0]
    dst = edge_index[1]
    pad = E_PAD - E
    src2d = jnp.concatenate(
        [src, jnp.zeros((pad,), jnp.int32)]).reshape(E_PAD // K, K)
    dst2d = jnp.concatenate(
        [dst, jnp.full((pad,), DUMMY, jnp.int32)]).reshape(E_PAD // K, K)

    x_pad = jnp.pad(feature, ((0, N_PAD - N), (0, 0)))
    ones_h = jnp.ones((K,), jnp.float32)
    zeros1 = jnp.zeros((N_PAD,), jnp.float32)
    zeros_h = jnp.zeros((N_PAD, H), jnp.float32)
    zeros_z = jnp.zeros((N_PAD, Z), jnp.float32)

    degp = _sc_deg(dst2d, ones_h, zeros1).reshape(NC, N_PAD)
    hs1, dinv = _tc_scale_mm1(x_pad, W1, degp)
    acc1 = _sc_gather_scatter(hs1, src2d, dst2d, zeros_h, H).reshape(NC, N_PAD, H)
    hs2 = _tc_relu_mm2(acc1, hs1, dinv, W_mu)
    acc2 = _sc_gather_scatter(hs2, src2d, dst2d, zeros_z, Z).reshape(NC, N_PAD, Z)
    mu = _tc_mu(acc2, hs2, dinv)
    out = _tc_decode(mu)
    return out[0, 0]


# speedup vs baseline: 1.1438x; 1.1438x over previous
import functools

import jax
import jax.numpy as jnp
from jax import lax
from jax.experimental import pallas as pl
from jax.experimental.pallas import tpu as pltpu
from jax.experimental.pallas import tpu_sc as plsc

N = 10000
D_IN = 128
H = 64
Z = 32
E = 320000

NC = 2
NS = 16
NW = NC * NS

K = 128
CH = 80
CH0 = 120
CH1 = 2 * CH - CH0
CHMX = max(CH0, CH1)
E_PAD = NW * CH * K
N_PAD = 10112
RPT = N_PAD // NS
DUMMY = N

def _mesh():
    return plsc.VectorSubcoreMesh(
        core_axis_name="c", subcore_axis_name="s", num_cores=NC, num_subcores=NS
    )


def _sc_deg(dst2d, ones_h, zeros_h):
    @functools.partial(
        pl.kernel,
        out_type=jax.ShapeDtypeStruct((NC, NS, RPT), jnp.float32),
        mesh=_mesh(),
        scratch_types=[
            pltpu.VMEM((CH, K), jnp.int32),
            pltpu.VMEM((K,), jnp.float32),
            pltpu.VMEM((RPT,), jnp.float32),
            pltpu.VMEM_SHARED((N_PAD,), jnp.float32),
        ],
        name="sc_deg",
    )
    def body(dst_h, ones_hbm, zeros_hbm, out_h, didx, ones_v, stage, acc):
        c = lax.axis_index("c")
        s = lax.axis_index("s")
        wid = s * NC + c
        pltpu.sync_copy(zeros_hbm.at[pl.ds(s * RPT, RPT)], stage)
        pltpu.sync_copy(stage, acc.at[pl.ds(s * RPT, RPT)])
        pltpu.sync_copy(ones_hbm, ones_v)
        pltpu.sync_copy(dst_h.at[pl.ds(wid * CH, CH)], didx)
        plsc.subcore_barrier()

        def step(j, carry):
            pltpu.sync_copy(ones_v, acc.at[didx.at[j]], add=True)
            return carry

        lax.fori_loop(0, CH, step, 0)
        plsc.subcore_barrier()
        pltpu.sync_copy(acc.at[pl.ds(s * RPT, RPT)], stage)
        pltpu.sync_copy(stage, out_h.at[c, s])

    return body(dst2d, ones_h, zeros_h)


def _sc_gather_scatter(hs, src2d, dst2d, zeros_h, d):
    @functools.partial(
        pl.kernel,
        out_type=jax.ShapeDtypeStruct((NC, NS, RPT, d), jnp.float32),
        mesh=_mesh(),
        scratch_types=[
            pltpu.VMEM((CHMX, K), jnp.int32),
            pltpu.VMEM((CHMX, K), jnp.int32),
            pltpu.VMEM((K, d), jnp.float32),
            pltpu.VMEM((K, d), jnp.float32),
            pltpu.VMEM((RPT, d), jnp.float32),
            pltpu.VMEM_SHARED((N_PAD, d), jnp.float32),
            pltpu.SemaphoreType.DMA,
            pltpu.SemaphoreType.DMA,
        ],
        compiler_params=pltpu.CompilerParams(use_tc_tiling_on_sc=False),
        name=f"sc_gs{d}",
    )
    def body(hs_h, src_h, dst_h, zeros_hbm, out_h,
             sidx, didx, rows0, rows1, stage, acc, sem0, sem1):
        c = lax.axis_index("c")
        s = lax.axis_index("s")
        pltpu.sync_copy(zeros_hbm.at[pl.ds(s * RPT, RPT)], stage)
        pltpu.sync_copy(stage, acc.at[pl.ds(s * RPT, RPT)])

        def run(ch, base):
            pltpu.sync_copy(src_h.at[pl.ds(base, ch)], sidx.at[pl.ds(0, ch)])
            pltpu.sync_copy(dst_h.at[pl.ds(base, ch)], didx.at[pl.ds(0, ch)])
            pltpu.async_copy(hs_h.at[sidx.at[0]], rows0, sem0)
            pltpu.async_copy(hs_h.at[sidx.at[1]], rows1, sem1)
            nh = ch // 2

            def step(jj, carry):
                j0 = 2 * jj

                def half(rows, sem, j):
                    pltpu.make_async_copy(hs_h.at[sidx.at[j]], rows, sem).wait()
                    pltpu.sync_copy(rows, acc.at[didx.at[j]], add=True)

                    @pl.when(jj < nh - 1)
                    def _():
                        pltpu.async_copy(hs_h.at[sidx.at[j + 2]], rows, sem)

                half(rows0, sem0, j0)
                half(rows1, sem1, j0 + 1)
                return carry

            lax.fori_loop(0, nh, step, 0)

        @pl.when(c == 0)
        def _():
            run(CH0, s * (CH0 + CH1))

        @pl.when(c == 1)
        def _():
            run(CH1, s * (CH0 + CH1) + CH0)

        plsc.subcore_barrier()
        pltpu.sync_copy(acc.at[pl.ds(s * RPT, RPT)], stage)
        pltpu.sync_copy(stage, out_h.at[c, s])

    return body(hs, src2d, dst2d, zeros_h)


_GB = 16
_BR = N_PAD // _GB


def _tc_scale_mm1(x_pad, w1, degp):
    def body(deg_ref, x_ref, w_ref, hs_ref, dinv_ref):
        deg = deg_ref[:, 0] + deg_ref[:, 1] + 1.0
        dinv = lax.rsqrt(jnp.maximum(deg, 1e-12))
        h = jnp.dot(x_ref[...], w_ref[...], preferred_element_type=jnp.float32)
        hs_ref[...] = h * dinv[:, None]
        dinv_ref[...] = dinv[:, None]

    return pl.pallas_call(
        body,
        grid=(_GB,),
        in_specs=[
            pl.BlockSpec((_BR, NC), lambda i: (i, 0)),
            pl.BlockSpec((_BR, D_IN), lambda i: (i, 0)),
            pl.BlockSpec((D_IN, H), lambda i: (0, 0)),
        ],
        out_specs=[
            pl.BlockSpec((_BR, H), lambda i: (i, 0)),
            pl.BlockSpec((_BR, 1), lambda i: (i, 0)),
        ],
        out_shape=[
            jax.ShapeDtypeStruct((N_PAD, H), jnp.float32),
            jax.ShapeDtypeStruct((N_PAD, 1), jnp.float32),
        ],
    )(degp.T, x_pad, w1)


def _tc_relu_mm2(acc1, hs1, dinv, w_mu):
    def body(a_ref, hs_ref, dinv_ref, w_ref, out_ref):
        dv = dinv_ref[...]
        h2 = jnp.maximum((a_ref[0] + a_ref[1] + hs_ref[...]) * dv, 0.0)
        out_ref[...] = (
            jnp.dot(h2, w_ref[...], preferred_element_type=jnp.float32) * dv
        )

    return pl.pallas_call(
        body,
        grid=(_GB,),
        in_specs=[
            pl.BlockSpec((NC, _BR, H), lambda i: (0, i, 0)),
            pl.BlockSpec((_BR, H), lambda i: (i, 0)),
            pl.BlockSpec((_BR, 1), lambda i: (i, 0)),
            pl.BlockSpec((H, Z), lambda i: (0, 0)),
        ],
        out_specs=pl.BlockSpec((_BR, Z), lambda i: (i, 0)),
        out_shape=jax.ShapeDtypeStruct((N_PAD, Z), jnp.float32),
    )(acc1, hs1, dinv, w_mu)


def _tc_mu(acc2, hs2, dinv):
    def body(a_ref, hs_ref, dinv_ref, out_ref):
        i = pl.program_id(0)
        mu = (a_ref[0] + a_ref[1] + hs_ref[...]) * dinv_ref[...]
        rows = lax.broadcasted_iota(jnp.int32, (_BR, Z), 0) + i * _BR
        out_ref[...] = jnp.where(rows < N, mu, 0.0)

    return pl.pallas_call(
        body,
        grid=(_GB,),
        in_specs=[
            pl.BlockSpec((NC, _BR, Z), lambda i: (0, i, 0)),
            pl.BlockSpec((_BR, Z), lambda i: (i, 0)),
            pl.BlockSpec((_BR, 1), lambda i: (i, 0)),
        ],
        out_specs=pl.BlockSpec((_BR, Z), lambda i: (i, 0)),
        out_shape=jax.ShapeDtypeStruct((N_PAD, Z), jnp.float32),
    )(acc2, hs2, dinv)


_PAD_CORR = 0.5 * float(N_PAD * N_PAD - N * N)


def _tc_decode(mu):
    def body(zi_ref, zj_ref, out_ref, acc_ref):
        i = pl.program_id(0)
        j = pl.program_id(1)

        @pl.when(jnp.logical_and(i == 0, j == 0))
        def _init():
            acc_ref[0, 0] = 0.0

        logits = lax.dot_general(
            zi_ref[...], zj_ref[...], (((1,), (1,)), ((), ())),
            preferred_element_type=jnp.float32,
        )
        acc_ref[0, 0] += jnp.sum(jax.nn.sigmoid(logits))

        @pl.when(jnp.logical_and(i == _GB - 1, j == _GB - 1))
        def _fin():
            out_ref[0, 0] = -(acc_ref[0, 0] - _PAD_CORR) / float(N * N)

    return pl.pallas_call(
        body,
        grid=(_GB, _GB),
        in_specs=[
            pl.BlockSpec((_BR, Z), lambda i, j: (i, 0)),
            pl.BlockSpec((_BR, Z), lambda i, j: (j, 0)),
        ],
        out_specs=pl.BlockSpec(memory_space=pltpu.SMEM),
        out_shape=jax.ShapeDtypeStruct((1, 1), jnp.float32),
        scratch_shapes=[pltpu.SMEM((1, 1), jnp.float32)],
    )(mu, mu)


@jax.jit
def kernel(feature, edge_index, W1, W_mu):
    src = edge_index[0]
    dst = edge_index[1]
    pad = E_PAD - E
    src2d = jnp.concatenate(
        [src, jnp.zeros((pad,), jnp.int32)]).reshape(E_PAD // K, K)
    dst2d = jnp.concatenate(
        [dst, jnp.full((pad,), DUMMY, jnp.int32)]).reshape(E_PAD // K, K)

    x_pad = jnp.pad(feature, ((0, N_PAD - N), (0, 0)))
    ones_h = jnp.ones((K,), jnp.float32)
    zeros1 = jnp.zeros((N_PAD,), jnp.float32)
    zeros_h = jnp.zeros((N_PAD, H), jnp.float32)
    zeros_z = jnp.zeros((N_PAD, Z), jnp.float32)

    degp = _sc_deg(dst2d, ones_h, zeros1).reshape(NC, N_PAD)
    hs1, dinv = _tc_scale_mm1(x_pad, W1, degp)
    acc1 = _sc_gather_scatter(hs1, src2d, dst2d, zeros_h, H).reshape(NC, N_PAD, H)
    hs2 = _tc_relu_mm2(acc1, hs1, dinv, W_mu)
    acc2 = _sc_gather_scatter(hs2, src2d, dst2d, zeros_z, Z).reshape(NC, N_PAD, Z)
    mu = _tc_mu(acc2, hs2, dinv)
    out = _tc_decode(mu)
    return out[0, 0]


# speedup vs baseline: 1.1592x; 1.0134x over previous
import functools

import jax
import jax.numpy as jnp
from jax import lax
from jax.experimental import pallas as pl
from jax.experimental.pallas import tpu as pltpu
from jax.experimental.pallas import tpu_sc as plsc

N = 10000
D_IN = 128
H = 64
Z = 32
E = 320000

NC = 2
NS = 16
NW = NC * NS

K = 128
CH = 80
CH0 = 120
CH1 = 2 * CH - CH0
CHMX = max(CH0, CH1)
E_PAD = NW * CH * K
N_PAD = 10112
RPT = N_PAD // NS
DUMMY = N

def _mesh():
    return plsc.VectorSubcoreMesh(
        core_axis_name="c", subcore_axis_name="s", num_cores=NC, num_subcores=NS
    )


def _sc_deg(dst2d, ones_h, zeros_h):
    @functools.partial(
        pl.kernel,
        out_type=jax.ShapeDtypeStruct((NC, NS, RPT), jnp.float32),
        mesh=_mesh(),
        scratch_types=[
            pltpu.VMEM((CH, K), jnp.int32),
            pltpu.VMEM((K,), jnp.float32),
            pltpu.VMEM((RPT,), jnp.float32),
            pltpu.VMEM_SHARED((N_PAD,), jnp.float32),
        ],
        name="sc_deg",
    )
    def body(dst_h, ones_hbm, zeros_hbm, out_h, didx, ones_v, stage, acc):
        c = lax.axis_index("c")
        s = lax.axis_index("s")
        wid = s * NC + c
        pltpu.sync_copy(zeros_hbm.at[pl.ds(s * RPT, RPT)], stage)
        pltpu.sync_copy(stage, acc.at[pl.ds(s * RPT, RPT)])
        pltpu.sync_copy(ones_hbm, ones_v)
        pltpu.sync_copy(dst_h.at[pl.ds(wid * CH, CH)], didx)
        plsc.subcore_barrier()

        def step(j, carry):
            pltpu.sync_copy(ones_v, acc.at[didx.at[j]], add=True)
            return carry

        lax.fori_loop(0, CH, step, 0)
        plsc.subcore_barrier()
        pltpu.sync_copy(acc.at[pl.ds(s * RPT, RPT)], stage)
        pltpu.sync_copy(stage, out_h.at[c, s])

    return body(dst2d, ones_h, zeros_h)


def _sc_gather_scatter(hs, src2d, dst2d, zeros_h, d):
    @functools.partial(
        pl.kernel,
        out_type=jax.ShapeDtypeStruct((NC, NS, RPT, d), jnp.float32),
        mesh=_mesh(),
        scratch_types=[
            pltpu.VMEM((CHMX, K), jnp.int32),
            pltpu.VMEM((CHMX, K), jnp.int32),
            pltpu.VMEM((K, d), jnp.float32),
            pltpu.VMEM((K, d), jnp.float32),
            pltpu.VMEM((RPT, d), jnp.float32),
            pltpu.VMEM_SHARED((N_PAD, d), jnp.float32),
            pltpu.SemaphoreType.DMA,
            pltpu.SemaphoreType.DMA,
        ],
        compiler_params=pltpu.CompilerParams(use_tc_tiling_on_sc=False),
        name=f"sc_gs{d}",
    )
    def body(hs_h, src_h, dst_h, zeros_hbm, out_h,
             sidx, didx, rows0, rows1, stage, acc, sem0, sem1):
        c = lax.axis_index("c")
        s = lax.axis_index("s")
        pltpu.sync_copy(zeros_hbm.at[pl.ds(s * RPT, RPT)], stage)
        pltpu.sync_copy(stage, acc.at[pl.ds(s * RPT, RPT)])

        def run(ch, base):
            pltpu.sync_copy(src_h.at[pl.ds(base, ch)], sidx.at[pl.ds(0, ch)])
            pltpu.sync_copy(dst_h.at[pl.ds(base, ch)], didx.at[pl.ds(0, ch)])
            pltpu.async_copy(hs_h.at[sidx.at[0]], rows0, sem0)
            pltpu.async_copy(hs_h.at[sidx.at[1]], rows1, sem1)
            nh = ch // 2

            def step(jj, carry):
                j0 = 2 * jj

                def half(rows, sem, j):
                    pltpu.make_async_copy(hs_h.at[sidx.at[j]], rows, sem).wait()
                    pltpu.sync_copy(rows, acc.at[didx.at[j]], add=True)

                    @pl.when(jj < nh - 1)
                    def _():
                        pltpu.async_copy(hs_h.at[sidx.at[j + 2]], rows, sem)

                half(rows0, sem0, j0)
                half(rows1, sem1, j0 + 1)
                return carry

            lax.fori_loop(0, nh, step, 0)

        @pl.when(c == 0)
        def _():
            run(CH0, s * (CH0 + CH1))

        @pl.when(c == 1)
        def _():
            run(CH1, s * (CH0 + CH1) + CH0)

        plsc.subcore_barrier()
        pltpu.sync_copy(acc.at[pl.ds(s * RPT, RPT)], stage)
        pltpu.sync_copy(stage, out_h.at[c, s])

    return body(hs, src2d, dst2d, zeros_h)


_GB = 16
_BR = N_PAD // _GB


def _tc_scale_mm1(x_pad, w1, degp):
    def body(deg_ref, x_ref, w_ref, hs_ref, dinv_ref):
        deg = deg_ref[:, 0] + deg_ref[:, 1] + 1.0
        dinv = lax.rsqrt(jnp.maximum(deg, 1e-12))
        h = jnp.dot(x_ref[...], w_ref[...], preferred_element_type=jnp.float32)
        hs_ref[...] = h * dinv[:, None]
        dinv_ref[...] = dinv[:, None]

    return pl.pallas_call(
        body,
        grid=(_GB,),
        in_specs=[
            pl.BlockSpec((_BR, NC), lambda i: (i, 0)),
            pl.BlockSpec((_BR, D_IN), lambda i: (i, 0)),
            pl.BlockSpec((D_IN, H), lambda i: (0, 0)),
        ],
        out_specs=[
            pl.BlockSpec((_BR, H), lambda i: (i, 0)),
            pl.BlockSpec((_BR, 1), lambda i: (i, 0)),
        ],
        out_shape=[
            jax.ShapeDtypeStruct((N_PAD, H), jnp.float32),
            jax.ShapeDtypeStruct((N_PAD, 1), jnp.float32),
        ],
    )(degp.T, x_pad, w1)


def _tc_relu_mm2(acc1, hs1, dinv, w_mu):
    def body(a_ref, hs_ref, dinv_ref, w_ref, out_ref):
        dv = dinv_ref[...]
        h2 = jnp.maximum((a_ref[0] + a_ref[1] + hs_ref[...]) * dv, 0.0)
        out_ref[...] = (
            jnp.dot(h2, w_ref[...], preferred_element_type=jnp.float32) * dv
        )

    return pl.pallas_call(
        body,
        grid=(_GB,),
        in_specs=[
            pl.BlockSpec((NC, _BR, H), lambda i: (0, i, 0)),
            pl.BlockSpec((_BR, H), lambda i: (i, 0)),
            pl.BlockSpec((_BR, 1), lambda i: (i, 0)),
            pl.BlockSpec((H, Z), lambda i: (0, 0)),
        ],
        out_specs=pl.BlockSpec((_BR, Z), lambda i: (i, 0)),
        out_shape=jax.ShapeDtypeStruct((N_PAD, Z), jnp.float32),
    )(acc1, hs1, dinv, w_mu)


def _tc_mu(acc2, hs2, dinv):
    def body(a_ref, hs_ref, dinv_ref, out_ref):
        i = pl.program_id(0)
        mu = (a_ref[0] + a_ref[1] + hs_ref[...]) * dinv_ref[...]
        rows = lax.broadcasted_iota(jnp.int32, (_BR, Z), 0) + i * _BR
        out_ref[...] = jnp.where(rows < N, mu, 0.0)

    return pl.pallas_call(
        body,
        grid=(_GB,),
        in_specs=[
            pl.BlockSpec((NC, _BR, Z), lambda i: (0, i, 0)),
            pl.BlockSpec((_BR, Z), lambda i: (i, 0)),
            pl.BlockSpec((_BR, 1), lambda i: (i, 0)),
        ],
        out_specs=pl.BlockSpec((_BR, Z), lambda i: (i, 0)),
        out_shape=jax.ShapeDtypeStruct((N_PAD, Z), jnp.float32),
    )(acc2, hs2, dinv)


def _tc_decode(mu):
    def body(zi_ref, zj_ref, out_ref, acc_ref):
        i = pl.program_id(0)
        j = pl.program_id(1)

        @pl.when(jnp.logical_and(i == 0, j == 0))
        def _init():
            acc_ref[0, 0] = 0.0

        logits = lax.dot_general(
            zi_ref[...], zj_ref[...], (((1,), (1,)), ((), ())),
            preferred_element_type=jnp.float32,
        )
        acc_ref[0, 0] += jnp.sum(jnp.tanh(0.5 * logits))

        @pl.when(jnp.logical_and(i == _GB - 1, j == _GB - 1))
        def _fin():
            out_ref[0, 0] = -0.5 - acc_ref[0, 0] / float(2 * N * N)

    return pl.pallas_call(
        body,
        grid=(_GB, _GB),
        in_specs=[
            pl.BlockSpec((_BR, Z), lambda i, j: (i, 0)),
            pl.BlockSpec((_BR, Z), lambda i, j: (j, 0)),
        ],
        out_specs=pl.BlockSpec(memory_space=pltpu.SMEM),
        out_shape=jax.ShapeDtypeStruct((1, 1), jnp.float32),
        scratch_shapes=[pltpu.SMEM((1, 1), jnp.float32)],
    )(mu, mu)


@jax.jit
def kernel(feature, edge_index, W1, W_mu):
    src = edge_index[0]
    dst = edge_index[1]
    pad = E_PAD - E
    src2d = jnp.concatenate(
        [src, jnp.zeros((pad,), jnp.int32)]).reshape(E_PAD // K, K)
    dst2d = jnp.concatenate(
        [dst, jnp.full((pad,), DUMMY, jnp.int32)]).reshape(E_PAD // K, K)

    x_pad = jnp.pad(feature, ((0, N_PAD - N), (0, 0)))
    ones_h = jnp.ones((K,), jnp.float32)
    zeros1 = jnp.zeros((N_PAD,), jnp.float32)
    zeros_h = jnp.zeros((N_PAD, H), jnp.float32)
    zeros_z = jnp.zeros((N_PAD, Z), jnp.float32)

    degp = _sc_deg(dst2d, ones_h, zeros1).reshape(NC, N_PAD)
    hs1, dinv = _tc_scale_mm1(x_pad, W1, degp)
    acc1 = _sc_gather_scatter(hs1, src2d, dst2d, zeros_h, H).reshape(NC, N_PAD, H)
    hs2 = _tc_relu_mm2(acc1, hs1, dinv, W_mu)
    acc2 = _sc_gather_scatter(hs2, src2d, dst2d, zeros_z, Z).reshape(NC, N_PAD, Z)
    mu = _tc_mu(acc2, hs2, dinv)
    out = _tc_decode(mu)
    return out[0, 0]


# speedup vs baseline: 1.4353x; 1.2382x over previous
import functools

import jax
import jax.numpy as jnp
from jax import lax
from jax.experimental import pallas as pl
from jax.experimental.pallas import tpu as pltpu
from jax.experimental.pallas import tpu_sc as plsc

N = 10000
D_IN = 128
H = 64
Z = 32
E = 320000

NC = 2
NS = 16
NW = NC * NS

K = 128
CH = 80
CH0 = 120
CH1 = 2 * CH - CH0
CHMX = max(CH0, CH1)
E_PAD = NW * CH * K
N_PAD = 10112
RPT = N_PAD // NS
DUMMY = N

def _mesh():
    return plsc.VectorSubcoreMesh(
        core_axis_name="c", subcore_axis_name="s", num_cores=NC, num_subcores=NS
    )


def _sc_deg(dst2d, ones_h, zeros_h):
    @functools.partial(
        pl.kernel,
        out_type=jax.ShapeDtypeStruct((NC, NS, RPT), jnp.float32),
        mesh=_mesh(),
        scratch_types=[
            pltpu.VMEM((CH, K), jnp.int32),
            pltpu.VMEM((K,), jnp.float32),
            pltpu.VMEM((RPT,), jnp.float32),
            pltpu.VMEM_SHARED((N_PAD,), jnp.float32),
        ],
        name="sc_deg",
    )
    def body(dst_h, ones_hbm, zeros_hbm, out_h, didx, ones_v, stage, acc):
        c = lax.axis_index("c")
        s = lax.axis_index("s")
        wid = s * NC + c
        pltpu.sync_copy(zeros_hbm.at[pl.ds(s * RPT, RPT)], stage)
        pltpu.sync_copy(stage, acc.at[pl.ds(s * RPT, RPT)])
        pltpu.sync_copy(ones_hbm, ones_v)
        pltpu.sync_copy(dst_h.at[pl.ds(wid * CH, CH)], didx)
        plsc.subcore_barrier()

        def step(j, carry):
            pltpu.sync_copy(ones_v, acc.at[didx.at[j]], add=True)
            return carry

        lax.fori_loop(0, CH, step, 0)
        plsc.subcore_barrier()
        pltpu.sync_copy(acc.at[pl.ds(s * RPT, RPT)], stage)
        pltpu.sync_copy(stage, out_h.at[c, s])

    return body(dst2d, ones_h, zeros_h)


def _sc_gather_scatter(hs, src2d, dst2d, zeros_h, d):
    @functools.partial(
        pl.kernel,
        out_type=jax.ShapeDtypeStruct((NC, NS, RPT, d), jnp.bfloat16),
        mesh=_mesh(),
        scratch_types=[
            pltpu.VMEM((CHMX, K), jnp.int32),
            pltpu.VMEM((CHMX, K), jnp.int32),
            pltpu.VMEM((K, d), jnp.bfloat16),
            pltpu.VMEM((K, d), jnp.bfloat16),
            pltpu.VMEM((RPT, d), jnp.bfloat16),
            pltpu.VMEM_SHARED((N_PAD, d), jnp.bfloat16),
            pltpu.SemaphoreType.DMA,
            pltpu.SemaphoreType.DMA,
        ],
        compiler_params=pltpu.CompilerParams(use_tc_tiling_on_sc=False),
        name=f"sc_gs{d}",
    )
    def body(hs_h, src_h, dst_h, zeros_hbm, out_h,
             sidx, didx, rows0, rows1, stage, acc, sem0, sem1):
        c = lax.axis_index("c")
        s = lax.axis_index("s")
        pltpu.sync_copy(zeros_hbm.at[pl.ds(s * RPT, RPT)], stage)
        pltpu.sync_copy(stage, acc.at[pl.ds(s * RPT, RPT)])

        def run(ch, base):
            pltpu.sync_copy(src_h.at[pl.ds(base, ch)], sidx.at[pl.ds(0, ch)])
            pltpu.sync_copy(dst_h.at[pl.ds(base, ch)], didx.at[pl.ds(0, ch)])
            pltpu.async_copy(hs_h.at[sidx.at[0]], rows0, sem0)
            pltpu.async_copy(hs_h.at[sidx.at[1]], rows1, sem1)
            nh = ch // 2

            def step(jj, carry):
                j0 = 2 * jj

                def half(rows, sem, j):
                    pltpu.make_async_copy(hs_h.at[sidx.at[j]], rows, sem).wait()
                    pltpu.sync_copy(rows, acc.at[didx.at[j]], add=True)

                    @pl.when(jj < nh - 1)
                    def _():
                        pltpu.async_copy(hs_h.at[sidx.at[j + 2]], rows, sem)

                half(rows0, sem0, j0)
                half(rows1, sem1, j0 + 1)
                return carry

            lax.fori_loop(0, nh, step, 0)

        @pl.when(c == 0)
        def _():
            run(CH0, s * (CH0 + CH1))

        @pl.when(c == 1)
        def _():
            run(CH1, s * (CH0 + CH1) + CH0)

        plsc.subcore_barrier()
        pltpu.sync_copy(acc.at[pl.ds(s * RPT, RPT)], stage)
        pltpu.sync_copy(stage, out_h.at[c, s])

    return body(hs, src2d, dst2d, zeros_h)


_GB = 16
_BR = N_PAD // _GB


def _tc_scale_mm1(x_pad, w1, degp):
    def body(deg_ref, x_ref, w_ref, hs_ref, dinv_ref):
        deg = deg_ref[:, 0] + deg_ref[:, 1] + 1.0
        dinv = lax.rsqrt(jnp.maximum(deg, 1e-12))
        h = jnp.dot(x_ref[...], w_ref[...], preferred_element_type=jnp.float32)
        hs_ref[...] = (h * dinv[:, None]).astype(jnp.bfloat16)
        dinv_ref[...] = dinv[:, None]

    return pl.pallas_call(
        body,
        grid=(_GB,),
        in_specs=[
            pl.BlockSpec((_BR, NC), lambda i: (i, 0)),
            pl.BlockSpec((_BR, D_IN), lambda i: (i, 0)),
            pl.BlockSpec((D_IN, H), lambda i: (0, 0)),
        ],
        out_specs=[
            pl.BlockSpec((_BR, H), lambda i: (i, 0)),
            pl.BlockSpec((_BR, 1), lambda i: (i, 0)),
        ],
        out_shape=[
            jax.ShapeDtypeStruct((N_PAD, H), jnp.bfloat16),
            jax.ShapeDtypeStruct((N_PAD, 1), jnp.float32),
        ],
    )(degp.T, x_pad, w1)


def _tc_relu_mm2(acc1, hs1, dinv, w_mu):
    def body(a_ref, hs_ref, dinv_ref, w_ref, out_ref):
        dv = dinv_ref[...]
        ssum = (a_ref[0].astype(jnp.float32) + a_ref[1].astype(jnp.float32)
                + hs_ref[...].astype(jnp.float32))
        h2 = jnp.maximum(ssum * dv, 0.0)
        out_ref[...] = (
            jnp.dot(h2, w_ref[...], preferred_element_type=jnp.float32) * dv
        ).astype(jnp.bfloat16)

    return pl.pallas_call(
        body,
        grid=(_GB,),
        in_specs=[
            pl.BlockSpec((NC, _BR, H), lambda i: (0, i, 0)),
            pl.BlockSpec((_BR, H), lambda i: (i, 0)),
            pl.BlockSpec((_BR, 1), lambda i: (i, 0)),
            pl.BlockSpec((H, Z), lambda i: (0, 0)),
        ],
        out_specs=pl.BlockSpec((_BR, Z), lambda i: (i, 0)),
        out_shape=jax.ShapeDtypeStruct((N_PAD, Z), jnp.bfloat16),
    )(acc1, hs1, dinv, w_mu)


def _tc_mu(acc2, hs2, dinv):
    def body(a_ref, hs_ref, dinv_ref, out_ref):
        i = pl.program_id(0)
        mu = (a_ref[0].astype(jnp.float32) + a_ref[1].astype(jnp.float32)
              + hs_ref[...].astype(jnp.float32)) * dinv_ref[...]
        rows = lax.broadcasted_iota(jnp.int32, (_BR, Z), 0) + i * _BR
        out_ref[...] = jnp.where(rows < N, mu, 0.0)

    return pl.pallas_call(
        body,
        grid=(_GB,),
        in_specs=[
            pl.BlockSpec((NC, _BR, Z), lambda i: (0, i, 0)),
            pl.BlockSpec((_BR, Z), lambda i: (i, 0)),
            pl.BlockSpec((_BR, 1), lambda i: (i, 0)),
        ],
        out_specs=pl.BlockSpec((_BR, Z), lambda i: (i, 0)),
        out_shape=jax.ShapeDtypeStruct((N_PAD, Z), jnp.float32),
    )(acc2, hs2, dinv)


def _tc_decode(mu):
    def body(zi_ref, zj_ref, out_ref, acc_ref):
        i = pl.program_id(0)
        j = pl.program_id(1)

        @pl.when(jnp.logical_and(i == 0, j == 0))
        def _init():
            acc_ref[0, 0] = 0.0

        logits = lax.dot_general(
            zi_ref[...], zj_ref[...], (((1,), (1,)), ((), ())),
            preferred_element_type=jnp.float32,
        )
        acc_ref[0, 0] += jnp.sum(jnp.tanh(0.5 * logits))

        @pl.when(jnp.logical_and(i == _GB - 1, j == _GB - 1))
        def _fin():
            out_ref[0, 0] = -0.5 - acc_ref[0, 0] / float(2 * N * N)

    return pl.pallas_call(
        body,
        grid=(_GB, _GB),
        in_specs=[
            pl.BlockSpec((_BR, Z), lambda i, j: (i, 0)),
            pl.BlockSpec((_BR, Z), lambda i, j: (j, 0)),
        ],
        out_specs=pl.BlockSpec(memory_space=pltpu.SMEM),
        out_shape=jax.ShapeDtypeStruct((1, 1), jnp.float32),
        scratch_shapes=[pltpu.SMEM((1, 1), jnp.float32)],
    )(mu, mu)


@jax.jit
def kernel(feature, edge_index, W1, W_mu):
    src = edge_index[0]
    dst = edge_index[1]
    pad = E_PAD - E
    src2d = jnp.concatenate(
        [src, jnp.zeros((pad,), jnp.int32)]).reshape(E_PAD // K, K)
    dst2d = jnp.concatenate(
        [dst, jnp.full((pad,), DUMMY, jnp.int32)]).reshape(E_PAD // K, K)

    x_pad = jnp.pad(feature, ((0, N_PAD - N), (0, 0)))
    ones_h = jnp.ones((K,), jnp.float32)
    zeros1 = jnp.zeros((N_PAD,), jnp.float32)
    zeros_h = jnp.zeros((N_PAD, H), jnp.bfloat16)
    zeros_z = jnp.zeros((N_PAD, Z), jnp.bfloat16)

    degp = _sc_deg(dst2d, ones_h, zeros1).reshape(NC, N_PAD)
    hs1, dinv = _tc_scale_mm1(x_pad, W1, degp)
    acc1 = _sc_gather_scatter(hs1, src2d, dst2d, zeros_h, H).reshape(NC, N_PAD, H)
    hs2 = _tc_relu_mm2(acc1, hs1, dinv, W_mu)
    acc2 = _sc_gather_scatter(hs2, src2d, dst2d, zeros_z, Z).reshape(NC, N_PAD, Z)
    mu = _tc_mu(acc2, hs2, dinv)
    out = _tc_decode(mu)
    return out[0, 0]


# speedup vs baseline: 1.5108x; 1.0526x over previous
import functools

import jax
import jax.numpy as jnp
from jax import lax
from jax.experimental import pallas as pl
from jax.experimental.pallas import tpu as pltpu
from jax.experimental.pallas import tpu_sc as plsc

N = 10000
D_IN = 128
H = 64
Z = 32
E = 320000

NC = 2
NS = 16
NW = NC * NS

K = 128
CH = 80
CH0 = 104
CH1 = 2 * CH - CH0
CHMX = max(CH0, CH1)
E_PAD = NW * CH * K
N_PAD = 10112
RPT = N_PAD // NS
DUMMY = N

def _mesh():
    return plsc.VectorSubcoreMesh(
        core_axis_name="c", subcore_axis_name="s", num_cores=NC, num_subcores=NS
    )


def _sc_deg(dst2d, ones_h, zeros_h):
    @functools.partial(
        pl.kernel,
        out_type=jax.ShapeDtypeStruct((NC, NS, RPT), jnp.float32),
        mesh=_mesh(),
        scratch_types=[
            pltpu.VMEM((CH, K), jnp.int32),
            pltpu.VMEM((K,), jnp.float32),
            pltpu.VMEM((RPT,), jnp.float32),
            pltpu.VMEM_SHARED((N_PAD,), jnp.float32),
        ],
        name="sc_deg",
    )
    def body(dst_h, ones_hbm, zeros_hbm, out_h, didx, ones_v, stage, acc):
        c = lax.axis_index("c")
        s = lax.axis_index("s")
        wid = s * NC + c
        pltpu.sync_copy(zeros_hbm.at[pl.ds(s * RPT, RPT)], stage)
        pltpu.sync_copy(stage, acc.at[pl.ds(s * RPT, RPT)])
        pltpu.sync_copy(ones_hbm, ones_v)
        pltpu.sync_copy(dst_h.at[pl.ds(wid * CH, CH)], didx)
        plsc.subcore_barrier()

        def step(j, carry):
            pltpu.sync_copy(ones_v, acc.at[didx.at[j]], add=True)
            return carry

        lax.fori_loop(0, CH, step, 0)
        plsc.subcore_barrier()
        pltpu.sync_copy(acc.at[pl.ds(s * RPT, RPT)], stage)
        pltpu.sync_copy(stage, out_h.at[c, s])

    return body(dst2d, ones_h, zeros_h)


def _sc_gather_scatter(hs, src2d, dst2d, zeros_h, d):
    @functools.partial(
        pl.kernel,
        out_type=jax.ShapeDtypeStruct((NC, NS, RPT, d), jnp.bfloat16),
        mesh=_mesh(),
        scratch_types=[
            pltpu.VMEM((CHMX, K), jnp.int32),
            pltpu.VMEM((CHMX, K), jnp.int32),
            pltpu.VMEM((K, d), jnp.bfloat16),
            pltpu.VMEM((K, d), jnp.bfloat16),
            pltpu.VMEM((RPT, d), jnp.bfloat16),
            pltpu.VMEM_SHARED((N_PAD, d), jnp.bfloat16),
            pltpu.SemaphoreType.DMA,
            pltpu.SemaphoreType.DMA,
        ],
        compiler_params=pltpu.CompilerParams(use_tc_tiling_on_sc=False),
        name=f"sc_gs{d}",
    )
    def body(hs_h, src_h, dst_h, zeros_hbm, out_h,
             sidx, didx, rows0, rows1, stage, acc, sem0, sem1):
        c = lax.axis_index("c")
        s = lax.axis_index("s")
        pltpu.sync_copy(zeros_hbm.at[pl.ds(s * RPT, RPT)], stage)
        pltpu.sync_copy(stage, acc.at[pl.ds(s * RPT, RPT)])

        def run(ch, base):
            pltpu.sync_copy(src_h.at[pl.ds(base, ch)], sidx.at[pl.ds(0, ch)])
            pltpu.sync_copy(dst_h.at[pl.ds(base, ch)], didx.at[pl.ds(0, ch)])
            pltpu.async_copy(hs_h.at[sidx.at[0]], rows0, sem0)
            pltpu.async_copy(hs_h.at[sidx.at[1]], rows1, sem1)
            nh = ch // 2

            def step(jj, carry):
                j0 = 2 * jj

                def half(rows, sem, j):
                    pltpu.make_async_copy(hs_h.at[sidx.at[j]], rows, sem).wait()
                    pltpu.sync_copy(rows, acc.at[didx.at[j]], add=True)

                    @pl.when(jj < nh - 1)
                    def _():
                        pltpu.async_copy(hs_h.at[sidx.at[j + 2]], rows, sem)

                half(rows0, sem0, j0)
                half(rows1, sem1, j0 + 1)
                return carry

            lax.fori_loop(0, nh, step, 0)

        @pl.when(c == 0)
        def _():
            run(CH0, s * (CH0 + CH1))

        @pl.when(c == 1)
        def _():
            run(CH1, s * (CH0 + CH1) + CH0)

        plsc.subcore_barrier()
        pltpu.sync_copy(acc.at[pl.ds(s * RPT, RPT)], stage)
        pltpu.sync_copy(stage, out_h.at[c, s])

    return body(hs, src2d, dst2d, zeros_h)


_GB = 16
_BR = N_PAD // _GB


def _tc_scale_mm1(x_pad, w1, degp):
    def body(deg_ref, x_ref, w_ref, hs_ref, dinv_ref):
        deg = deg_ref[:, 0] + deg_ref[:, 1] + 1.0
        dinv = lax.rsqrt(jnp.maximum(deg, 1e-12))
        h = jnp.dot(x_ref[...], w_ref[...], preferred_element_type=jnp.float32)
        hs_ref[...] = (h * dinv[:, None]).astype(jnp.bfloat16)
        dinv_ref[...] = dinv[:, None]

    return pl.pallas_call(
        body,
        grid=(_GB,),
        in_specs=[
            pl.BlockSpec((_BR, NC), lambda i: (i, 0)),
            pl.BlockSpec((_BR, D_IN), lambda i: (i, 0)),
            pl.BlockSpec((D_IN, H), lambda i: (0, 0)),
        ],
        out_specs=[
            pl.BlockSpec((_BR, H), lambda i: (i, 0)),
            pl.BlockSpec((_BR, 1), lambda i: (i, 0)),
        ],
        out_shape=[
            jax.ShapeDtypeStruct((N_PAD, H), jnp.bfloat16),
            jax.ShapeDtypeStruct((N_PAD, 1), jnp.float32),
        ],
    )(degp.T, x_pad, w1)


def _tc_relu_mm2(acc1, hs1, dinv, w_mu):
    def body(a_ref, hs_ref, dinv_ref, w_ref, out_ref):
        dv = dinv_ref[...]
        ssum = (a_ref[0].astype(jnp.float32) + a_ref[1].astype(jnp.float32)
                + hs_ref[...].astype(jnp.float32))
        h2 = jnp.maximum(ssum * dv, 0.0)
        out_ref[...] = (
            jnp.dot(h2, w_ref[...], preferred_element_type=jnp.float32) * dv
        ).astype(jnp.bfloat16)

    return pl.pallas_call(
        body,
        grid=(_GB,),
        in_specs=[
            pl.BlockSpec((NC, _BR, H), lambda i: (0, i, 0)),
            pl.BlockSpec((_BR, H), lambda i: (i, 0)),
            pl.BlockSpec((_BR, 1), lambda i: (i, 0)),
            pl.BlockSpec((H, Z), lambda i: (0, 0)),
        ],
        out_specs=pl.BlockSpec((_BR, Z), lambda i: (i, 0)),
        out_shape=jax.ShapeDtypeStruct((N_PAD, Z), jnp.bfloat16),
    )(acc1, hs1, dinv, w_mu)


def _tc_mu(acc2, hs2, dinv):
    def body(a_ref, hs_ref, dinv_ref, out_ref):
        i = pl.program_id(0)
        mu = (a_ref[0].astype(jnp.float32) + a_ref[1].astype(jnp.float32)
              + hs_ref[...].astype(jnp.float32)) * dinv_ref[...]
        rows = lax.broadcasted_iota(jnp.int32, (_BR, Z), 0) + i * _BR
        out_ref[...] = jnp.where(rows < N, mu, 0.0).astype(jnp.bfloat16)

    return pl.pallas_call(
        body,
        grid=(_GB,),
        in_specs=[
            pl.BlockSpec((NC, _BR, Z), lambda i: (0, i, 0)),
            pl.BlockSpec((_BR, Z), lambda i: (i, 0)),
            pl.BlockSpec((_BR, 1), lambda i: (i, 0)),
        ],
        out_specs=pl.BlockSpec((_BR, Z), lambda i: (i, 0)),
        out_shape=jax.ShapeDtypeStruct((N_PAD, Z), jnp.bfloat16),
    )(acc2, hs2, dinv)


def _tc_decode(mu):
    def body(zi_ref, zj_ref, out_ref, acc_ref):
        i = pl.program_id(0)
        j = pl.program_id(1)

        @pl.when(jnp.logical_and(i == 0, j == 0))
        def _init():
            acc_ref[0, 0] = 0.0

        logits = lax.dot_general(
            zi_ref[...], zj_ref[...], (((1,), (1,)), ((), ())),
            preferred_element_type=jnp.float32,
        )
        t = jnp.tanh((0.5 * logits).astype(jnp.bfloat16))
        acc_ref[0, 0] += jnp.sum(t.astype(jnp.float32))

        @pl.when(jnp.logical_and(i == _GB - 1, j == _GB - 1))
        def _fin():
            out_ref[0, 0] = -0.5 - acc_ref[0, 0] / float(2 * N * N)

    return pl.pallas_call(
        body,
        grid=(_GB, _GB),
        in_specs=[
            pl.BlockSpec((_BR, Z), lambda i, j: (i, 0)),
            pl.BlockSpec((_BR, Z), lambda i, j: (j, 0)),
        ],
        out_specs=pl.BlockSpec(memory_space=pltpu.SMEM),
        out_shape=jax.ShapeDtypeStruct((1, 1), jnp.float32),
        scratch_shapes=[pltpu.SMEM((1, 1), jnp.float32)],
    )(mu, mu)


@jax.jit
def kernel(feature, edge_index, W1, W_mu):
    src = edge_index[0]
    dst = edge_index[1]
    pad = E_PAD - E
    src2d = jnp.concatenate(
        [src, jnp.zeros((pad,), jnp.int32)]).reshape(E_PAD // K, K)
    dst2d = jnp.concatenate(
        [dst, jnp.full((pad,), DUMMY, jnp.int32)]).reshape(E_PAD // K, K)

    x_pad = jnp.pad(feature, ((0, N_PAD - N), (0, 0)))
    ones_h = jnp.ones((K,), jnp.float32)
    zeros1 = jnp.zeros((N_PAD,), jnp.float32)
    zeros_h = jnp.zeros((N_PAD, H), jnp.bfloat16)
    zeros_z = jnp.zeros((N_PAD, Z), jnp.bfloat16)

    degp = _sc_deg(dst2d, ones_h, zeros1).reshape(NC, N_PAD)
    hs1, dinv = _tc_scale_mm1(x_pad, W1, degp)
    acc1 = _sc_gather_scatter(hs1, src2d, dst2d, zeros_h, H).reshape(NC, N_PAD, H)
    hs2 = _tc_relu_mm2(acc1, hs1, dinv, W_mu)
    acc2 = _sc_gather_scatter(hs2, src2d, dst2d, zeros_z, Z).reshape(NC, N_PAD, Z)
    mu = _tc_mu(acc2, hs2, dinv)
    out = _tc_decode(mu)
    return out[0, 0]


# speedup vs baseline: 1.6619x; 1.1000x over previous
import functools

import jax
import jax.numpy as jnp
from jax import lax
from jax.experimental import pallas as pl
from jax.experimental.pallas import tpu as pltpu
from jax.experimental.pallas import tpu_sc as plsc

N = 10000
D_IN = 128
H = 64
Z = 32
E = 320000

NC = 2
NS = 16
NW = NC * NS

K = 128
CH = 80
CH0 = 104
CH1 = 2 * CH - CH0
CHMX = max(CH0, CH1)
E_PAD = NW * CH * K
N_PAD = 10112
RPT = N_PAD // NS
DUMMY = N

def _mesh():
    return plsc.VectorSubcoreMesh(
        core_axis_name="c", subcore_axis_name="s", num_cores=NC, num_subcores=NS
    )


def _sc_deg(dst2d, ones_h, zeros_h):
    @functools.partial(
        pl.kernel,
        out_type=jax.ShapeDtypeStruct((NC, NS, RPT), jnp.float32),
        mesh=_mesh(),
        scratch_types=[
            pltpu.VMEM((CH, K), jnp.int32),
            pltpu.VMEM((K,), jnp.float32),
            pltpu.VMEM((RPT,), jnp.float32),
            pltpu.VMEM_SHARED((N_PAD,), jnp.float32),
        ],
        name="sc_deg",
    )
    def body(dst_h, ones_hbm, zeros_hbm, out_h, didx, ones_v, stage, acc):
        c = lax.axis_index("c")
        s = lax.axis_index("s")
        wid = s * NC + c
        pltpu.sync_copy(zeros_hbm.at[pl.ds(s * RPT, RPT)], stage)
        pltpu.sync_copy(stage, acc.at[pl.ds(s * RPT, RPT)])
        pltpu.sync_copy(ones_hbm, ones_v)
        pltpu.sync_copy(dst_h.at[pl.ds(wid * CH, CH)], didx)
        plsc.subcore_barrier()

        def step(j, carry):
            pltpu.sync_copy(ones_v, acc.at[didx.at[j]], add=True)
            return carry

        lax.fori_loop(0, CH, step, 0)
        plsc.subcore_barrier()
        pltpu.sync_copy(acc.at[pl.ds(s * RPT, RPT)], stage)
        pltpu.sync_copy(stage, out_h.at[c, s])

    return body(dst2d, ones_h, zeros_h)


def _sc_gather_scatter(hs, src2d, dst2d, zeros_h, d):
    @functools.partial(
        pl.kernel,
        out_type=jax.ShapeDtypeStruct((NC, NS, RPT, d), jnp.bfloat16),
        mesh=_mesh(),
        scratch_types=[
            pltpu.VMEM((CHMX, K), jnp.int32),
            pltpu.VMEM((CHMX, K), jnp.int32),
            pltpu.VMEM((K, d), jnp.bfloat16),
            pltpu.VMEM((K, d), jnp.bfloat16),
            pltpu.VMEM((RPT, d), jnp.bfloat16),
            pltpu.VMEM_SHARED((N_PAD, d), jnp.bfloat16),
            pltpu.SemaphoreType.DMA,
            pltpu.SemaphoreType.DMA,
        ],
        compiler_params=pltpu.CompilerParams(use_tc_tiling_on_sc=False),
        name=f"sc_gs{d}",
    )
    def body(hs_h, src_h, dst_h, zeros_hbm, out_h,
             sidx, didx, rows0, rows1, stage, acc, sem0, sem1):
        c = lax.axis_index("c")
        s = lax.axis_index("s")
        pltpu.sync_copy(zeros_hbm.at[pl.ds(s * RPT, RPT)], stage)
        pltpu.sync_copy(stage, acc.at[pl.ds(s * RPT, RPT)])

        def run(ch, base):
            pltpu.sync_copy(src_h.at[pl.ds(base, ch)], sidx.at[pl.ds(0, ch)])
            pltpu.sync_copy(dst_h.at[pl.ds(base, ch)], didx.at[pl.ds(0, ch)])
            pltpu.async_copy(hs_h.at[sidx.at[0]], rows0, sem0)
            pltpu.async_copy(hs_h.at[sidx.at[1]], rows1, sem1)
            nh = ch // 2

            def step(jj, carry):
                j0 = 2 * jj

                def half(rows, sem, j):
                    pltpu.make_async_copy(hs_h.at[sidx.at[j]], rows, sem).wait()
                    pltpu.sync_copy(rows, acc.at[didx.at[j]], add=True)

                    @pl.when(jj < nh - 1)
                    def _():
                        pltpu.async_copy(hs_h.at[sidx.at[j + 2]], rows, sem)

                half(rows0, sem0, j0)
                half(rows1, sem1, j0 + 1)
                return carry

            lax.fori_loop(0, nh, step, 0)

        @pl.when(c == 0)
        def _():
            run(CH0, s * (CH0 + CH1))

        @pl.when(c == 1)
        def _():
            run(CH1, s * (CH0 + CH1) + CH0)

        plsc.subcore_barrier()
        pltpu.sync_copy(acc.at[pl.ds(s * RPT, RPT)], stage)
        pltpu.sync_copy(stage, out_h.at[c, s])

    return body(hs, src2d, dst2d, zeros_h)


_GB = 16
_BR = N_PAD // _GB


def _tc_scale_mm1(x_pad, w1, degp):
    def body(deg_ref, x_ref, w_ref, hs_ref, dinv_ref):
        deg = deg_ref[:, 0] + deg_ref[:, 1] + 1.0
        dinv = lax.rsqrt(jnp.maximum(deg, 1e-12))
        h = jnp.dot(x_ref[...], w_ref[...], preferred_element_type=jnp.float32)
        hs_ref[...] = (h * dinv[:, None]).astype(jnp.bfloat16)
        dinv_ref[...] = dinv[:, None]

    return pl.pallas_call(
        body,
        grid=(_GB,),
        in_specs=[
            pl.BlockSpec((_BR, NC), lambda i: (i, 0)),
            pl.BlockSpec((_BR, D_IN), lambda i: (i, 0)),
            pl.BlockSpec((D_IN, H), lambda i: (0, 0)),
        ],
        out_specs=[
            pl.BlockSpec((_BR, H), lambda i: (i, 0)),
            pl.BlockSpec((_BR, 1), lambda i: (i, 0)),
        ],
        out_shape=[
            jax.ShapeDtypeStruct((N_PAD, H), jnp.bfloat16),
            jax.ShapeDtypeStruct((N_PAD, 1), jnp.float32),
        ],
    )(degp.T, x_pad, w1)


def _tc_relu_mm2(acc1, hs1, dinv, w_mu):
    def body(a_ref, hs_ref, dinv_ref, w_ref, out_ref):
        dv = dinv_ref[...]
        ssum = (a_ref[0].astype(jnp.float32) + a_ref[1].astype(jnp.float32)
                + hs_ref[...].astype(jnp.float32))
        h2 = jnp.maximum(ssum * dv, 0.0)
        out_ref[...] = (
            jnp.dot(h2, w_ref[...], preferred_element_type=jnp.float32) * dv
        ).astype(jnp.bfloat16)

    return pl.pallas_call(
        body,
        grid=(_GB,),
        in_specs=[
            pl.BlockSpec((NC, _BR, H), lambda i: (0, i, 0)),
            pl.BlockSpec((_BR, H), lambda i: (i, 0)),
            pl.BlockSpec((_BR, 1), lambda i: (i, 0)),
            pl.BlockSpec((H, Z), lambda i: (0, 0)),
        ],
        out_specs=pl.BlockSpec((_BR, Z), lambda i: (i, 0)),
        out_shape=jax.ShapeDtypeStruct((N_PAD, Z), jnp.bfloat16),
    )(acc1, hs1, dinv, w_mu)


def _tc_mu(acc2, hs2, dinv):
    def body(a_ref, hs_ref, dinv_ref, out_ref):
        i = pl.program_id(0)
        mu = (a_ref[0].astype(jnp.float32) + a_ref[1].astype(jnp.float32)
              + hs_ref[...].astype(jnp.float32)) * dinv_ref[...]
        rows = lax.broadcasted_iota(jnp.int32, (_BR, Z), 0) + i * _BR
        out_ref[...] = jnp.where(rows < N, mu, 0.0).astype(jnp.bfloat16)

    return pl.pallas_call(
        body,
        grid=(_GB,),
        in_specs=[
            pl.BlockSpec((NC, _BR, Z), lambda i: (0, i, 0)),
            pl.BlockSpec((_BR, Z), lambda i: (i, 0)),
            pl.BlockSpec((_BR, 1), lambda i: (i, 0)),
        ],
        out_specs=pl.BlockSpec((_BR, Z), lambda i: (i, 0)),
        out_shape=jax.ShapeDtypeStruct((N_PAD, Z), jnp.bfloat16),
    )(acc2, hs2, dinv)


def _tc_decode(mu):
    def body(zi_ref, zj_ref, out_ref, acc_ref):
        i = pl.program_id(0)
        j = pl.program_id(1)

        @pl.when(jnp.logical_and(i == 0, j == 0))
        def _init():
            acc_ref[0, 0] = 0.0

        @pl.when(j >= i)
        def _work():
            logits = lax.dot_general(
                zi_ref[...], zj_ref[...], (((1,), (1,)), ((), ())),
                preferred_element_type=jnp.float32,
            )
            t = jnp.tanh((0.5 * logits).astype(jnp.bfloat16))
            w = jnp.where(j > i, 2.0, 1.0)
            acc_ref[0, 0] += w * jnp.sum(t.astype(jnp.float32))

        @pl.when(jnp.logical_and(i == _GB - 1, j == _GB - 1))
        def _fin():
            out_ref[0, 0] = -0.5 - acc_ref[0, 0] / float(2 * N * N)

    return pl.pallas_call(
        body,
        grid=(_GB, _GB),
        in_specs=[
            pl.BlockSpec((_BR, Z), lambda i, j: (i, 0)),
            pl.BlockSpec((_BR, Z), lambda i, j: (j, 0)),
        ],
        out_specs=pl.BlockSpec(memory_space=pltpu.SMEM),
        out_shape=jax.ShapeDtypeStruct((1, 1), jnp.float32),
        scratch_shapes=[pltpu.SMEM((1, 1), jnp.float32)],
    )(mu, mu)


@jax.jit
def kernel(feature, edge_index, W1, W_mu):
    src = edge_index[0]
    dst = edge_index[1]
    pad = E_PAD - E
    src2d = jnp.concatenate(
        [src, jnp.zeros((pad,), jnp.int32)]).reshape(E_PAD // K, K)
    dst2d = jnp.concatenate(
        [dst, jnp.full((pad,), DUMMY, jnp.int32)]).reshape(E_PAD // K, K)

    x_pad = jnp.pad(feature, ((0, N_PAD - N), (0, 0)))
    ones_h = jnp.ones((K,), jnp.float32)
    zeros1 = jnp.zeros((N_PAD,), jnp.float32)
    zeros_h = jnp.zeros((N_PAD, H), jnp.bfloat16)
    zeros_z = jnp.zeros((N_PAD, Z), jnp.bfloat16)

    degp = _sc_deg(dst2d, ones_h, zeros1).reshape(NC, N_PAD)
    hs1, dinv = _tc_scale_mm1(x_pad, W1, degp)
    acc1 = _sc_gather_scatter(hs1, src2d, dst2d, zeros_h, H).reshape(NC, N_PAD, H)
    hs2 = _tc_relu_mm2(acc1, hs1, dinv, W_mu)
    acc2 = _sc_gather_scatter(hs2, src2d, dst2d, zeros_z, Z).reshape(NC, N_PAD, Z)
    mu = _tc_mu(acc2, hs2, dinv)
    out = _tc_decode(mu)
    return out[0, 0]


# speedup vs baseline: 2.0258x; 1.2190x over previous
import functools

import jax
import jax.numpy as jnp
from jax import lax
from jax.experimental import pallas as pl
from jax.experimental.pallas import tpu as pltpu
from jax.experimental.pallas import tpu_sc as plsc

N = 10000
D_IN = 128
H = 64
Z = 32
E = 320000

NC = 2
NS = 16
NW = NC * NS

K = 128
CH = 80
CH0 = 104
CH1 = 2 * CH - CH0
CHMX = max(CH0, CH1)
E_PAD = NW * CH * K
N_PAD = 10112
RPT = N_PAD // NS
DUMMY = N

def _mesh():
    return plsc.VectorSubcoreMesh(
        core_axis_name="c", subcore_axis_name="s", num_cores=NC, num_subcores=NS
    )


def _sc_deg(dst2d, ones_h, zeros_h):
    @functools.partial(
        pl.kernel,
        out_type=jax.ShapeDtypeStruct((NC, NS, RPT), jnp.float32),
        mesh=_mesh(),
        scratch_types=[
            pltpu.VMEM((CH, K), jnp.int32),
            pltpu.VMEM((K,), jnp.float32),
            pltpu.VMEM((RPT,), jnp.float32),
            pltpu.VMEM_SHARED((N_PAD,), jnp.float32),
        ],
        name="sc_deg",
    )
    def body(dst_h, ones_hbm, zeros_hbm, out_h, didx, ones_v, stage, acc):
        c = lax.axis_index("c")
        s = lax.axis_index("s")
        wid = s * NC + c
        pltpu.sync_copy(zeros_hbm.at[pl.ds(s * RPT, RPT)], stage)
        pltpu.sync_copy(stage, acc.at[pl.ds(s * RPT, RPT)])
        pltpu.sync_copy(ones_hbm, ones_v)
        pltpu.sync_copy(dst_h.at[pl.ds(wid * CH, CH)], didx)
        plsc.subcore_barrier()

        def step(j, carry):
            pltpu.sync_copy(ones_v, acc.at[didx.at[j]], add=True)
            return carry

        lax.fori_loop(0, CH, step, 0)
        plsc.subcore_barrier()
        pltpu.sync_copy(acc.at[pl.ds(s * RPT, RPT)], stage)
        pltpu.sync_copy(stage, out_h.at[c, s])

    return body(dst2d, ones_h, zeros_h)


def _sc_gather_scatter(hs, src2d, dst2d, zeros_h, d):
    @functools.partial(
        pl.kernel,
        out_type=jax.ShapeDtypeStruct((NC, NS, RPT, d), jnp.bfloat16),
        mesh=_mesh(),
        scratch_types=[
            pltpu.VMEM((CHMX, K), jnp.int32),
            pltpu.VMEM((CHMX, K), jnp.int32),
            pltpu.VMEM((K, d), jnp.bfloat16),
            pltpu.VMEM((K, d), jnp.bfloat16),
            pltpu.VMEM((RPT, d), jnp.bfloat16),
            pltpu.VMEM_SHARED((N_PAD, d), jnp.bfloat16),
            pltpu.SemaphoreType.DMA,
            pltpu.SemaphoreType.DMA,
        ],
        compiler_params=pltpu.CompilerParams(use_tc_tiling_on_sc=False),
        name=f"sc_gs{d}",
    )
    def body(hs_h, src_h, dst_h, zeros_hbm, out_h,
             sidx, didx, rows0, rows1, stage, acc, sem0, sem1):
        c = lax.axis_index("c")
        s = lax.axis_index("s")
        pltpu.sync_copy(zeros_hbm.at[pl.ds(s * RPT, RPT)], stage)
        pltpu.sync_copy(stage, acc.at[pl.ds(s * RPT, RPT)])

        def run(ch, base):
            pltpu.sync_copy(src_h.at[pl.ds(base, ch)], sidx.at[pl.ds(0, ch)])
            pltpu.sync_copy(dst_h.at[pl.ds(base, ch)], didx.at[pl.ds(0, ch)])
            pltpu.async_copy(hs_h.at[sidx.at[0]], rows0, sem0)
            pltpu.async_copy(hs_h.at[sidx.at[1]], rows1, sem1)
            nh = ch // 2

            def step(jj, carry):
                j0 = 2 * jj

                def half(rows, sem, j):
                    pltpu.make_async_copy(hs_h.at[sidx.at[j]], rows, sem).wait()
                    pltpu.sync_copy(rows, acc.at[didx.at[j]], add=True)

                    @pl.when(jj < nh - 1)
                    def _():
                        pltpu.async_copy(hs_h.at[sidx.at[j + 2]], rows, sem)

                half(rows0, sem0, j0)
                half(rows1, sem1, j0 + 1)
                return carry

            lax.fori_loop(0, nh, step, 0)

        @pl.when(c == 0)
        def _():
            run(CH0, s * (CH0 + CH1))

        @pl.when(c == 1)
        def _():
            run(CH1, s * (CH0 + CH1) + CH0)

        plsc.subcore_barrier()
        pltpu.sync_copy(acc.at[pl.ds(s * RPT, RPT)], stage)
        pltpu.sync_copy(stage, out_h.at[c, s])

    return body(hs, src2d, dst2d, zeros_h)


_GB = 16
_BR = N_PAD // _GB


def _tc_scale_mm1(x_pad, w1, degp):
    def body(deg_ref, x_ref, w_ref, hs_ref, dinv_ref):
        deg = deg_ref[:, 0] + deg_ref[:, 1] + 1.0
        dinv = lax.rsqrt(jnp.maximum(deg, 1e-12))
        h = jnp.dot(x_ref[...], w_ref[...], preferred_element_type=jnp.float32)
        hs_ref[...] = (h * dinv[:, None]).astype(jnp.bfloat16)
        dinv_ref[...] = dinv[:, None]

    return pl.pallas_call(
        body,
        grid=(_GB,),
        in_specs=[
            pl.BlockSpec((_BR, NC), lambda i: (i, 0)),
            pl.BlockSpec((_BR, D_IN), lambda i: (i, 0)),
            pl.BlockSpec((D_IN, H), lambda i: (0, 0)),
        ],
        out_specs=[
            pl.BlockSpec((_BR, H), lambda i: (i, 0)),
            pl.BlockSpec((_BR, 1), lambda i: (i, 0)),
        ],
        out_shape=[
            jax.ShapeDtypeStruct((N_PAD, H), jnp.bfloat16),
            jax.ShapeDtypeStruct((N_PAD, 1), jnp.float32),
        ],
    )(degp.T, x_pad, w1)


def _tc_relu_mm2(acc1, hs1, dinv, w_mu):
    def body(a_ref, hs_ref, dinv_ref, w_ref, out_ref):
        dv = dinv_ref[...]
        ssum = (a_ref[0].astype(jnp.float32) + a_ref[1].astype(jnp.float32)
                + hs_ref[...].astype(jnp.float32))
        h2 = jnp.maximum(ssum * dv, 0.0)
        out_ref[...] = (
            jnp.dot(h2, w_ref[...], preferred_element_type=jnp.float32) * dv
        ).astype(jnp.bfloat16)

    return pl.pallas_call(
        body,
        grid=(_GB,),
        in_specs=[
            pl.BlockSpec((NC, _BR, H), lambda i: (0, i, 0)),
            pl.BlockSpec((_BR, H), lambda i: (i, 0)),
            pl.BlockSpec((_BR, 1), lambda i: (i, 0)),
            pl.BlockSpec((H, Z), lambda i: (0, 0)),
        ],
        out_specs=pl.BlockSpec((_BR, Z), lambda i: (i, 0)),
        out_shape=jax.ShapeDtypeStruct((N_PAD, Z), jnp.bfloat16),
    )(acc1, hs1, dinv, w_mu)


def _tc_mu(acc2, hs2, dinv):
    def body(a_ref, hs_ref, dinv_ref, out_ref):
        i = pl.program_id(0)
        mu = (a_ref[0].astype(jnp.float32) + a_ref[1].astype(jnp.float32)
              + hs_ref[...].astype(jnp.float32)) * dinv_ref[...]
        rows = lax.broadcasted_iota(jnp.int32, (_BR, Z), 0) + i * _BR
        out_ref[...] = jnp.where(rows < N, mu, 0.0).astype(jnp.bfloat16)

    return pl.pallas_call(
        body,
        grid=(_GB,),
        in_specs=[
            pl.BlockSpec((NC, _BR, Z), lambda i: (0, i, 0)),
            pl.BlockSpec((_BR, Z), lambda i: (i, 0)),
            pl.BlockSpec((_BR, 1), lambda i: (i, 0)),
        ],
        out_specs=pl.BlockSpec((_BR, Z), lambda i: (i, 0)),
        out_shape=jax.ShapeDtypeStruct((N_PAD, Z), jnp.bfloat16),
    )(acc2, hs2, dinv)


def _tc_decode(mu):
    GBD = 8
    BD = N_PAD // GBD
    HALF = GBD // 2 + 1

    def body(zi_ref, zj_ref, out_ref, acc_ref):
        i = pl.program_id(0)
        jj = pl.program_id(1)

        @pl.when(jnp.logical_and(i == 0, jj == 0))
        def _init():
            acc_ref[0, 0] = 0.0

        logits = lax.dot_general(
            zi_ref[...], zj_ref[...], (((1,), (1,)), ((), ())),
            preferred_element_type=jnp.float32,
        )
        t = jnp.tanh((0.5 * logits).astype(jnp.bfloat16))
        w = jnp.where(jnp.logical_or(jj == 0, jj == HALF - 1), 1.0, 2.0)
        acc_ref[0, 0] += w * jnp.sum(t.astype(jnp.float32))

        @pl.when(jnp.logical_and(i == GBD - 1, jj == HALF - 1))
        def _fin():
            out_ref[0, 0] = -0.5 - acc_ref[0, 0] / float(2 * N * N)

    return pl.pallas_call(
        body,
        grid=(GBD, HALF),
        in_specs=[
            pl.BlockSpec((BD, Z), lambda i, jj: (i, 0)),
            pl.BlockSpec((BD, Z), lambda i, jj: ((i + jj) % GBD, 0)),
        ],
        out_specs=pl.BlockSpec(memory_space=pltpu.SMEM),
        out_shape=jax.ShapeDtypeStruct((1, 1), jnp.float32),
        scratch_shapes=[pltpu.SMEM((1, 1), jnp.float32)],
    )(mu, mu)


@jax.jit
def kernel(feature, edge_index, W1, W_mu):
    src = edge_index[0]
    dst = edge_index[1]
    pad = E_PAD - E
    src2d = jnp.concatenate(
        [src, jnp.zeros((pad,), jnp.int32)]).reshape(E_PAD // K, K)
    dst2d = jnp.concatenate(
        [dst, jnp.full((pad,), DUMMY, jnp.int32)]).reshape(E_PAD // K, K)

    x_pad = jnp.pad(feature, ((0, N_PAD - N), (0, 0)))
    ones_h = jnp.ones((K,), jnp.float32)
    zeros1 = jnp.zeros((N_PAD,), jnp.float32)
    zeros_h = jnp.zeros((N_PAD, H), jnp.bfloat16)
    zeros_z = jnp.zeros((N_PAD, Z), jnp.bfloat16)

    degp = _sc_deg(dst2d, ones_h, zeros1).reshape(NC, N_PAD)
    hs1, dinv = _tc_scale_mm1(x_pad, W1, degp)
    acc1 = _sc_gather_scatter(hs1, src2d, dst2d, zeros_h, H).reshape(NC, N_PAD, H)
    hs2 = _tc_relu_mm2(acc1, hs1, dinv, W_mu)
    acc2 = _sc_gather_scatter(hs2, src2d, dst2d, zeros_z, Z).reshape(NC, N_PAD, Z)
    mu = _tc_mu(acc2, hs2, dinv)
    out = _tc_decode(mu)
    return out[0, 0]
